# Initial kernel scaffold; baseline (speedup 1.0000x reference)
import jax
import jax.numpy as jnp
from jax.experimental import pallas as pl


def kernel(predictions, targets, anchors):
    raise NotImplementedError("write your pallas kernel here")


# speedup vs baseline: 2.2389x; 2.2389x over previous
import jax
import jax.numpy as jnp
from jax.experimental import pallas as pl

B, A, S, C = 64, 3, 52, 80
NC = 5 + C
N = B * A * S * S
LANES = 128
ROWS = N // LANES
GRID = 39
RB = ROWS // GRID


def _loss_kernel(pred_ref, tgt_ref, an_ref, out_ref):
    pid = pl.program_id(0)
    ident = (jax.lax.broadcasted_iota(jnp.int32, (LANES, LANES), 0)
             == jax.lax.broadcasted_iota(jnp.int32, (LANES, LANES), 1)
             ).astype(jnp.float32)
    dn = (((1,), (0,)), ((), ()))
    pt = jax.lax.dot_general(pred_ref[:, :, :8], ident, dn,
                             preferred_element_type=jnp.float32)
    tt = jax.lax.dot_general(tgt_ref[:, :, :8], ident, dn,
                             preferred_element_type=jnp.float32)
    p0 = pt[:, 0, :]
    t0 = tt[:, 0, :]
    t1 = tt[:, 1, :]
    t2 = tt[:, 2, :]
    t3 = tt[:, 3, :]
    t4 = tt[:, 4, :]

    row = jax.lax.broadcasted_iota(jnp.int32, (RB, LANES), 0) + pid * RB
    lane = jax.lax.broadcasted_iota(jnp.int32, (RB, LANES), 1)
    cell = row * LANES + lane
    x = (cell % S).astype(jnp.float32)
    y = ((cell // S) % S).astype(jnp.float32)
    a_idx = (cell // (S * S)) % A

    obj_m = (t0 == 1.0).astype(jnp.float32)
    noobj_m = (t0 == 0.0).astype(jnp.float32)

    noobj_terms = (jnp.maximum(p0, 0.0) - p0 * t0
                   + jnp.log1p(jnp.exp(-jnp.abs(p0))))

    aw = jnp.where(a_idx == 0, an_ref[0, 0],
                   jnp.where(a_idx == 1, an_ref[1, 0], an_ref[2, 0]))
    ah = jnp.where(a_idx == 0, an_ref[0, 1],
                   jnp.where(a_idx == 1, an_ref[1, 1], an_ref[2, 1]))
    bx = jax.nn.sigmoid(t1) + x
    by = jax.nn.sigmoid(t2) + y
    bw = jnp.exp(t3) * aw
    bh = jnp.exp(t4) * ah

    b1x1 = bx - bw * 0.5
    b1y1 = by - bh * 0.5
    b1x2 = bx + bw * 0.5
    b1y2 = by + bh * 0.5
    b2x1 = t1 - t3 * 0.5
    b2y1 = t2 - t4 * 0.5
    b2x2 = t1 + t3 * 0.5
    b2y2 = t2 + t4 * 0.5
    ix1 = jnp.maximum(b1x1, b2x1)
    iy1 = jnp.maximum(b1y1, b2y1)
    ix2 = jnp.minimum(b1x2, b2x2)
    iy2 = jnp.minimum(b1y2, b2y2)
    inter = (jnp.clip(ix2 - ix1, 0.0, None) * jnp.clip(iy2 - iy1, 0.0, None))
    area1 = (b1x2 - b1x1) * (b1y2 - b1y1)
    area2 = (b2x2 - b2x1) * (b2y2 - b2y1)
    union = area1 + area2 - inter + 1e-6
    iou = inter / union
    obj_terms = (jnp.maximum(iou, 0.0) - iou * p0
                 + jnp.log1p(jnp.exp(-jnp.abs(iou))))

    noobj_row = jnp.sum(noobj_terms * noobj_m, axis=0, keepdims=True)
    obj_row = jnp.sum(obj_terms * obj_m, axis=0, keepdims=True)
    k_row = jnp.sum(obj_m, axis=0, keepdims=True)
    n_row = jnp.sum(noobj_m, axis=0, keepdims=True)
    zero = jnp.zeros((4, LANES), dtype=jnp.float32)
    out_ref[...] = jnp.concatenate(
        [noobj_row, obj_row, k_row, n_row, zero], axis=0)


@jax.jit
def kernel(predictions, targets, anchors):
    pred_r = predictions.reshape(ROWS, LANES, NC)
    tgt_r = targets.reshape(ROWS, LANES, NC)
    anch = jnp.zeros((8, 128), jnp.float32).at[:A, :2].set(anchors)

    partials = pl.pallas_call(
        _loss_kernel,
        grid=(GRID,),
        in_specs=[
            pl.BlockSpec((RB, LANES, NC), lambda i: (i, 0, 0)),
            pl.BlockSpec((RB, LANES, NC), lambda i: (i, 0, 0)),
            pl.BlockSpec((8, 128), lambda i: (0, 0)),
        ],
        out_specs=pl.BlockSpec((None, 8, 128), lambda i: (i, 0, 0)),
        out_shape=jax.ShapeDtypeStruct((GRID, 8, 128), jnp.float32),
    )(pred_r, tgt_r, anch)

    sums = jnp.sum(partials, axis=(0, 2))
    no_obj_loss = sums[0] / sums[3]
    obj_loss = sums[1] / sums[2]
    return 0.5 * no_obj_loss + obj_loss
